# baseline (speedup 1.0000x reference)
import functools

import jax
import jax.numpy as jnp
from jax import lax
from jax.experimental import pallas as pl
from jax.experimental.pallas import tpu as pltpu
from jax.experimental.pallas import tpu_sc as plsc

_NC = 2
_NS = 16
_NW = _NC * _NS
_CH = 128


def _make_seg_sum(n_nodes, width, chunks_per_tile):
  acc_rows = (n_nodes // 128 + 1) * 128
  zchunk = acc_rows // _NS
  nhalf = chunks_per_tile // 2
  mesh = plsc.VectorSubcoreMesh(core_axis_name="c", subcore_axis_name="s")

  @functools.partial(
      pl.kernel,
      out_type=jax.ShapeDtypeStruct((_NC, acc_rows, width), jnp.float32),
      mesh=mesh,
      scratch_types=[
          pltpu.VMEM_SHARED((acc_rows, width), jnp.float32),
          pltpu.VMEM((chunks_per_tile + 1, _CH), jnp.int32),
          pltpu.VMEM((chunks_per_tile + 1, _CH), jnp.int32),
          pltpu.VMEM((_CH, width), jnp.float32),
          pltpu.VMEM((_CH, width), jnp.float32),
          pltpu.SemaphoreType.DMA,
          pltpu.SemaphoreType.DMA,
      ],
      compiler_params=pltpu.CompilerParams(use_tc_tiling_on_sc=False),
  )
  def seg_kernel(table, srcb, dstb, zrows, out, acc,
                 src_v, dst_v, rows0, rows1, sem0, sem1):
    c = lax.axis_index("c")
    s = lax.axis_index("s")
    wid = c * _NS + s
    pltpu.sync_copy(zrows, acc.at[pl.ds(s * zchunk, zchunk)])
    pltpu.sync_copy(srcb.at[wid], src_v)
    pltpu.sync_copy(dstb.at[wid], dst_v)
    plsc.subcore_barrier()

    pltpu.async_copy(table.at[src_v.at[0]], rows0, sem0)

    def body(i, carry):
      j0 = 2 * i
      pltpu.async_copy(table.at[src_v.at[j0 + 1]], rows1, sem1)
      pltpu.make_async_copy(table.at[src_v.at[j0]], rows0, sem0).wait()
      pltpu.sync_copy(rows0, acc.at[dst_v.at[j0]], add=True)
      pltpu.async_copy(table.at[src_v.at[j0 + 2]], rows0, sem0)
      pltpu.make_async_copy(table.at[src_v.at[j0]], rows1, sem1).wait()
      pltpu.sync_copy(rows1, acc.at[dst_v.at[j0 + 1]], add=True)
      return carry

    lax.fori_loop(0, nhalf, body, 0)
    pltpu.make_async_copy(table.at[src_v.at[0]], rows0, sem0).wait()
    plsc.subcore_barrier()
    pltpu.sync_copy(acc.at[pl.ds(s * zchunk, zchunk)],
                    out.at[c, pl.ds(s * zchunk, zchunk)])

  return seg_kernel


def _lin_body(x_ref, w_ref, b_ref, o_ref):
  o_ref[...] = lax.dot_general(
      x_ref[...], w_ref[...], (((1,), (1,)), ((), ())),
      preferred_element_type=jnp.float32) + b_ref[...]


def _mid_body(s0_ref, s1_ref, c0_ref, c1_ref, q_ref, w_ref, b_ref, o_ref):
  cnt = jnp.maximum(c0_ref[...] + c1_ref[...], 1.0)
  h = jnp.maximum((s0_ref[...] + s1_ref[...]) / cnt + q_ref[...], 0.0)
  o_ref[...] = lax.dot_general(
      h, w_ref[...], (((1,), (1,)), ((), ())),
      preferred_element_type=jnp.float32) + b_ref[...]


def _out_body(s0_ref, s1_ref, c0_ref, c1_ref, q_ref, o_ref):
  cnt = jnp.maximum(c0_ref[...] + c1_ref[...], 1.0)
  z = (s0_ref[...] + s1_ref[...]) / cnt + q_ref[...]
  z = z - jnp.max(z, axis=1, keepdims=True)
  o_ref[...] = z - jnp.log(jnp.sum(jnp.exp(z), axis=1, keepdims=True))


def kernel(x, edge_index, W1l, b1l, W1r, W2l, b2l, W2r):
  n = x.shape[0]
  e = edge_index.shape[1]
  hid = W1l.shape[0]
  out_ch = W2l.shape[0]

  src = edge_index[0].astype(jnp.int32)
  dst = edge_index[1].astype(jnp.int32)
  per = _NW * _CH
  chunks = -(-e // per)
  chunks += chunks % 2
  pad = chunks * per - e
  srcb = jnp.concatenate([src, jnp.zeros((pad,), jnp.int32)]).reshape(
      _NW, chunks, _CH)
  srcb = jnp.concatenate([srcb, jnp.zeros((_NW, 1, _CH), jnp.int32)], axis=1)
  dstb = jnp.concatenate([dst, jnp.full((pad,), n, jnp.int32)]).reshape(
      _NW, chunks, _CH)
  dstb = jnp.concatenate([dstb, jnp.full((_NW, 1, _CH), n, jnp.int32)], axis=1)

  w1 = jnp.concatenate([W1l, W1r], axis=0)
  bias1 = jnp.concatenate([jnp.zeros((hid,), jnp.float32), b1l])[None, :]
  pq1 = pl.pallas_call(
      _lin_body,
      out_shape=jax.ShapeDtypeStruct((n, 2 * hid), jnp.float32),
  )(x, w1, bias1)

  w_tab = 2 * hid
  table1 = jnp.concatenate(
      [pq1[:, :hid], jnp.ones((n, 1), jnp.float32),
       jnp.zeros((n, w_tab - hid - 1), jnp.float32)], axis=1)

  npad = (n // 128 + 1) * 128
  zrows32 = jnp.zeros((npad // _NS, w_tab), jnp.float32)
  seg32 = _make_seg_sum(n, w_tab, chunks)
  part1 = seg32(table1, srcb, dstb, zrows32)

  s0 = part1[0, :n, :hid]
  s1 = part1[1, :n, :hid]
  c0 = part1[0, :n, hid:hid + 1]
  c1 = part1[1, :n, hid:hid + 1]

  w2 = jnp.concatenate([W2l, W2r], axis=0)
  bias2 = jnp.concatenate([jnp.zeros((out_ch,), jnp.float32), b2l])[None, :]
  pq2 = pl.pallas_call(
      _mid_body,
      out_shape=jax.ShapeDtypeStruct((n, 2 * out_ch), jnp.float32),
  )(s0, s1, c0, c1, pq1[:, hid:], w2, bias2)

  table2 = pq2[:, :out_ch]
  zrows16 = jnp.zeros((npad // _NS, out_ch), jnp.float32)
  seg16 = _make_seg_sum(n, out_ch, chunks)
  part2 = seg16(table2, srcb, dstb, zrows16)

  out = pl.pallas_call(
      _out_body,
      out_shape=jax.ShapeDtypeStruct((n, out_ch), jnp.float32),
  )(part2[0, :n], part2[1, :n], c0, c1, pq2[:, out_ch:])
  return out


# speedup vs baseline: 1.2459x; 1.2459x over previous
import functools

import jax
import jax.numpy as jnp
from jax import lax
from jax.experimental import pallas as pl
from jax.experimental.pallas import tpu as pltpu
from jax.experimental.pallas import tpu_sc as plsc

_NC = 2
_NS = 16
_NW = _NC * _NS
_CH = 128


def _make_seg_sum(n_nodes, width, chunks_per_tile, with_counts):
  acc_rows = (n_nodes // 128 + 1) * 128
  zchunk = acc_rows // _NS
  cnt_rows = acc_rows
  cchunk = cnt_rows // _NS
  mesh = plsc.VectorSubcoreMesh(core_axis_name="c", subcore_axis_name="s")

  out_type = jax.ShapeDtypeStruct((_NC, acc_rows, width), jnp.float32)
  if with_counts:
    out_type = [out_type,
                jax.ShapeDtypeStruct((_NC, cnt_rows), jnp.float32)]
  scratch = [
      pltpu.VMEM_SHARED((acc_rows, width), jnp.float32),
      pltpu.VMEM((chunks_per_tile, _CH), jnp.int32),
      pltpu.VMEM((chunks_per_tile, _CH), jnp.int32),
      pltpu.VMEM((_CH, width), jnp.float32),
      pltpu.SemaphoreType.DMA,
  ]
  if with_counts:
    scratch.append(pltpu.VMEM_SHARED((cnt_rows,), jnp.float32))
    scratch.append(pltpu.VMEM((_CH,), jnp.float32))

  @functools.partial(
      pl.kernel,
      out_type=out_type,
      mesh=mesh,
      scratch_types=scratch,
      compiler_params=pltpu.CompilerParams(use_tc_tiling_on_sc=False),
  )
  def seg_kernel(table, srcb, dstb, zrows, *rest):
    if with_counts:
      zcnt, out, cnt_out, acc, src_v, dst_v, rows_v, sem, cnt_acc, ones_v = rest
    else:
      out, acc, src_v, dst_v, rows_v, sem = rest
    c = lax.axis_index("c")
    s = lax.axis_index("s")
    wid = c * _NS + s
    pltpu.sync_copy(zrows, acc.at[pl.ds(s * zchunk, zchunk)])
    if with_counts:
      pltpu.sync_copy(zcnt, cnt_acc.at[pl.ds(s * cchunk, cchunk)])
      for i in range(_CH // 16):
        ones_v[pl.ds(i * 16, 16)] = jnp.ones((16,), jnp.float32)
    pltpu.sync_copy(srcb.at[wid], src_v)
    pltpu.sync_copy(dstb.at[wid], dst_v)
    plsc.subcore_barrier()

    def body(j, carry):
      pltpu.async_copy(table.at[src_v.at[j]], rows_v, sem).wait()
      pltpu.sync_copy(rows_v, acc.at[dst_v.at[j]], add=True)
      if with_counts:
        pltpu.sync_copy(ones_v, cnt_acc.at[dst_v.at[j]], add=True)
      return carry

    lax.fori_loop(0, chunks_per_tile, body, 0)
    plsc.subcore_barrier()
    pltpu.sync_copy(acc.at[pl.ds(s * zchunk, zchunk)],
                    out.at[c, pl.ds(s * zchunk, zchunk)])
    if with_counts:
      pltpu.sync_copy(cnt_acc.at[pl.ds(s * cchunk, cchunk)],
                      cnt_out.at[c, pl.ds(s * cchunk, cchunk)])

  return seg_kernel


def _lin_body(x_ref, w_ref, b_ref, o_ref):
  o_ref[...] = lax.dot_general(
      x_ref[...], w_ref[...], (((1,), (1,)), ((), ())),
      preferred_element_type=jnp.float32) + b_ref[...]


def _mid_body(s0_ref, s1_ref, c0_ref, c1_ref, q_ref, w_ref, b_ref, o_ref):
  cnt = jnp.maximum(c0_ref[...] + c1_ref[...], 1.0)
  h = jnp.maximum((s0_ref[...] + s1_ref[...]) / cnt + q_ref[...], 0.0)
  o_ref[...] = lax.dot_general(
      h, w_ref[...], (((1,), (1,)), ((), ())),
      preferred_element_type=jnp.float32) + b_ref[...]


def _out_body(s0_ref, s1_ref, c0_ref, c1_ref, q_ref, o_ref):
  cnt = jnp.maximum(c0_ref[...] + c1_ref[...], 1.0)
  z = (s0_ref[...] + s1_ref[...]) / cnt + q_ref[...]
  z = z - jnp.max(z, axis=1, keepdims=True)
  o_ref[...] = z - jnp.log(jnp.sum(jnp.exp(z), axis=1, keepdims=True))


def kernel(x, edge_index, W1l, b1l, W1r, W2l, b2l, W2r):
  n = x.shape[0]
  e = edge_index.shape[1]
  hid = W1l.shape[0]
  out_ch = W2l.shape[0]

  src = edge_index[0].astype(jnp.int32)
  dst = edge_index[1].astype(jnp.int32)
  per = _NW * _CH
  chunks = -(-e // per)
  pad = chunks * per - e
  srcb = jnp.concatenate([src, jnp.zeros((pad,), jnp.int32)]).reshape(
      _NW, chunks, _CH)
  dstb = jnp.concatenate([dst, jnp.full((pad,), n, jnp.int32)]).reshape(
      _NW, chunks, _CH)

  npad = (n // 128 + 1) * 128
  zrows = jnp.zeros((npad // _NS, hid), jnp.float32)
  zcnt = jnp.zeros((npad // _NS,), jnp.float32)

  w1 = jnp.concatenate([W1l, W1r], axis=0)
  bias1 = jnp.concatenate([jnp.zeros((hid,), jnp.float32), b1l])[None, :]
  pq1 = pl.pallas_call(
      _lin_body,
      out_shape=jax.ShapeDtypeStruct((n, 2 * hid), jnp.float32),
  )(x, w1, bias1)

  seg_c = _make_seg_sum(n, hid, chunks, True)
  part1, cnts = seg_c(pq1[:, :hid], srcb, dstb, zrows, zcnt)

  s0 = part1[0, :n]
  s1 = part1[1, :n]
  c0 = cnts[0, :n, None]
  c1 = cnts[1, :n, None]

  w2 = jnp.concatenate([W2l, W2r], axis=0)
  bias2 = jnp.concatenate([jnp.zeros((out_ch,), jnp.float32), b2l])[None, :]
  pq2 = pl.pallas_call(
      _mid_body,
      out_shape=jax.ShapeDtypeStruct((n, 2 * out_ch), jnp.float32),
  )(s0, s1, c0, c1, pq1[:, hid:], w2, bias2)

  seg_p = _make_seg_sum(n, out_ch, chunks, False)
  part2 = seg_p(pq2[:, :out_ch], srcb, dstb, zrows)

  out = pl.pallas_call(
      _out_body,
      out_shape=jax.ShapeDtypeStruct((n, out_ch), jnp.float32),
  )(part2[0, :n], part2[1, :n], c0, c1, pq2[:, out_ch:])
  return out


# speedup vs baseline: 1.2951x; 1.0395x over previous
import functools

import jax
import jax.numpy as jnp
from jax import lax
from jax.experimental import pallas as pl
from jax.experimental.pallas import tpu as pltpu
from jax.experimental.pallas import tpu_sc as plsc

_NC = 2
_NS = 16
_NW = _NC * _NS
_CH = 128


def _acc_rows(n_nodes):
  return (n_nodes // 256 + 1) * 256


def _seg_loop(table, src_v, dst_v, rows_v, sem, acc, chunks, cnt_pair=None):
  def body(j, carry):
    pltpu.async_copy(table.at[src_v.at[j]], rows_v, sem).wait()
    pltpu.sync_copy(rows_v, acc.at[dst_v.at[j]], add=True)
    if cnt_pair is not None:
      ones_v, cnt_acc = cnt_pair
      pltpu.sync_copy(ones_v, cnt_acc.at[dst_v.at[j]], add=True)
    return carry

  lax.fori_loop(0, chunks, body, 0)


def _make_seg1(n_nodes, width, chunks_per_tile):
  acc_rows = _acc_rows(n_nodes)
  zchunk = acc_rows // _NS
  mesh = plsc.VectorSubcoreMesh(core_axis_name="c", subcore_axis_name="s")

  @functools.partial(
      pl.kernel,
      out_type=[jax.ShapeDtypeStruct((_NC, acc_rows, width), jnp.float32),
                jax.ShapeDtypeStruct((_NC, acc_rows), jnp.float32)],
      mesh=mesh,
      scratch_types=[
          pltpu.VMEM_SHARED((acc_rows, width), jnp.float32),
          pltpu.VMEM_SHARED((acc_rows,), jnp.float32),
          pltpu.VMEM((chunks_per_tile, _CH), jnp.int32),
          pltpu.VMEM((chunks_per_tile, _CH), jnp.int32),
          pltpu.VMEM((_CH, width), jnp.float32),
          pltpu.VMEM((_CH,), jnp.float32),
          pltpu.SemaphoreType.DMA,
      ],
      compiler_params=pltpu.CompilerParams(use_tc_tiling_on_sc=False, needs_layout_passes=False),
  )
  def seg1(table, srcb, dstb, zrows, zcnt, out, cnt_out,
           acc, cnt_acc, src_v, dst_v, rows_v, ones_v, sem):
    c = lax.axis_index("c")
    s = lax.axis_index("s")
    wid = c * _NS + s
    pltpu.sync_copy(zrows, acc.at[pl.ds(s * zchunk, zchunk)])
    pltpu.sync_copy(zcnt, cnt_acc.at[pl.ds(s * zchunk, zchunk)])
    for i in range(_CH // 16):
      ones_v[pl.ds(i * 16, 16)] = jnp.ones((16,), jnp.float32)
    pltpu.sync_copy(srcb.at[wid], src_v)
    pltpu.sync_copy(dstb.at[wid], dst_v)
    plsc.subcore_barrier()
    _seg_loop(table, src_v, dst_v, rows_v, sem, acc, chunks_per_tile,
              cnt_pair=(ones_v, cnt_acc))
    plsc.subcore_barrier()
    pltpu.sync_copy(acc.at[pl.ds(s * zchunk, zchunk)],
                    out.at[c, pl.ds(s * zchunk, zchunk)])
    pltpu.sync_copy(cnt_acc.at[pl.ds(s * zchunk, zchunk)],
                    cnt_out.at[c, pl.ds(s * zchunk, zchunk)])

  return seg1


def _make_seg2(n_nodes, width, chunks_per_tile):
  acc_rows = _acc_rows(n_nodes)
  zchunk = acc_rows // _NS
  mesh = plsc.VectorSubcoreMesh(core_axis_name="c", subcore_axis_name="s")

  @functools.partial(
      pl.kernel,
      out_type=[jax.ShapeDtypeStruct((_NC, acc_rows, width), jnp.float32),
                jax.ShapeDtypeStruct((_NC, acc_rows, width), jnp.float32)],
      mesh=mesh,
      scratch_types=[
          pltpu.VMEM_SHARED((acc_rows, width), jnp.float32),
          pltpu.VMEM((chunks_per_tile, _CH), jnp.int32),
          pltpu.VMEM((chunks_per_tile, _CH), jnp.int32),
          pltpu.VMEM((_CH, width), jnp.float32),
          pltpu.VMEM((zchunk, width), jnp.float32),
          pltpu.VMEM((zchunk, width), jnp.float32),
          pltpu.VMEM((zchunk, width), jnp.float32),
          pltpu.VMEM((zchunk,), jnp.float32),
          pltpu.VMEM((zchunk,), jnp.float32),
          pltpu.SemaphoreType.DMA,
      ],
      compiler_params=pltpu.CompilerParams(use_tc_tiling_on_sc=False, needs_layout_passes=False),
  )
  def seg2(part1, cnts, q1p, srcb, dstb, zrows, out, h_out,
           acc, src_v, dst_v, rows_v, s0_v, s1_v, q_v, cnt_v, cnt1_v, sem):
    c = lax.axis_index("c")
    s = lax.axis_index("s")
    wid = c * _NS + s
    sl = pl.ds(s * zchunk, zchunk)
    pltpu.sync_copy(zrows, acc.at[sl])
    pltpu.sync_copy(srcb.at[wid], src_v)
    pltpu.sync_copy(dstb.at[wid], dst_v)
    pltpu.sync_copy(part1.at[0, sl], s0_v)
    pltpu.sync_copy(part1.at[1, sl], s1_v)
    pltpu.sync_copy(q1p.at[sl], q_v)
    pltpu.sync_copy(cnts.at[0, sl], cnt_v)
    pltpu.sync_copy(cnts.at[1, sl], cnt1_v)

    def cbody(k, carry):
      d = pl.ds(k * 16, 16)
      cnt_v[d] = jnp.maximum(cnt_v[d] + cnt1_v[d], 1.0)
      return carry

    lax.fori_loop(0, zchunk // 16, cbody, 0)

    def hbody(i, carry):
      bc = plsc.load_gather(cnt_v, [jnp.full((16,), i, jnp.int32)])
      s0_v[i] = jnp.maximum((s0_v[i] + s1_v[i]) / bc + q_v[i], 0.0)
      return carry

    lax.fori_loop(0, zchunk, hbody, 0)
    pltpu.sync_copy(s0_v, h_out.at[c, sl])
    plsc.subcore_barrier()
    _seg_loop(h_out.at[c], src_v, dst_v, rows_v, sem, acc, chunks_per_tile)
    plsc.subcore_barrier()
    pltpu.sync_copy(acc.at[sl], out.at[c, sl])

  return seg2


def _lin_body(x_ref, w_ref, b_ref, o_ref):
  o_ref[...] = lax.dot_general(
      x_ref[...], w_ref[...], (((1,), (1,)), ((), ())),
      preferred_element_type=jnp.float32) + b_ref[...]


def _out_body(s0_ref, s1_ref, c0_ref, c1_ref, h_ref, w_ref, b_ref, o_ref):
  cnt = jnp.maximum(c0_ref[...] + c1_ref[...], 1.0)
  mean2 = (s0_ref[...] + s1_ref[...]) / cnt
  z = lax.dot_general(
      jnp.concatenate([mean2, h_ref[...]], axis=1), w_ref[...],
      (((1,), (1,)), ((), ())), preferred_element_type=jnp.float32) + b_ref[...]
  z = z - jnp.max(z, axis=1, keepdims=True)
  o_ref[...] = z - jnp.log(jnp.sum(jnp.exp(z), axis=1, keepdims=True))


def kernel(x, edge_index, W1l, b1l, W1r, W2l, b2l, W2r):
  n = x.shape[0]
  e = edge_index.shape[1]
  hid = W1l.shape[0]
  out_ch = W2l.shape[0]

  src = edge_index[0].astype(jnp.int32)
  dst = edge_index[1].astype(jnp.int32)
  per = _NW * _CH
  chunks = -(-e // per)
  pad = chunks * per - e
  srcb = jnp.concatenate([src, jnp.zeros((pad,), jnp.int32)]).reshape(
      _NW, chunks, _CH)
  dstb = jnp.concatenate([dst, jnp.full((pad,), n, jnp.int32)]).reshape(
      _NW, chunks, _CH)

  npad = _acc_rows(n)
  zrows = jnp.zeros((npad // _NS, hid), jnp.float32)
  zcnt = jnp.zeros((npad // _NS,), jnp.float32)

  w1 = jnp.concatenate([W1l, W1r], axis=0)
  bias1 = jnp.concatenate([jnp.zeros((hid,), jnp.float32), b1l])[None, :]
  pq1 = pl.pallas_call(
      _lin_body,
      out_shape=jax.ShapeDtypeStruct((n, 2 * hid), jnp.float32),
  )(x, w1, bias1)

  part1, cnts = _make_seg1(n, hid, chunks)(pq1[:, :hid], srcb, dstb,
                                           zrows, zcnt)

  q1p = jnp.concatenate(
      [pq1[:, hid:], jnp.zeros((npad - n, hid), jnp.float32)])
  part2, h_out = _make_seg2(n, out_ch, chunks)(part1, cnts, q1p,
                                               srcb, dstb, zrows)

  w2 = jnp.concatenate([W2l, W2r], axis=1)
  c0 = cnts[0, :n, None]
  c1 = cnts[1, :n, None]
  out = pl.pallas_call(
      _out_body,
      out_shape=jax.ShapeDtypeStruct((n, out_ch), jnp.float32),
  )(part2[0, :n], part2[1, :n], c0, c1, h_out[0, :n], w2, b2l[None, :])
  return out


# speedup vs baseline: 1.3829x; 1.0678x over previous
import functools

import jax
import jax.numpy as jnp
from jax import lax
from jax.experimental import pallas as pl
from jax.experimental.pallas import tpu as pltpu
from jax.experimental.pallas import tpu_sc as plsc

_NC = 2
_NS = 16
_NW = _NC * _NS
_CH = 128


def _acc_rows(n_nodes):
  return (n_nodes // 256 + 1) * 256


def _seg_loop(table, src_v, dst_v, rows_v, sem, acc, chunks, cnt_pair=None):
  def body(j, carry):
    pltpu.async_copy(table.at[src_v.at[j]], rows_v, sem).wait()
    pltpu.sync_copy(rows_v, acc.at[dst_v.at[j]], add=True)
    if cnt_pair is not None:
      ones_v, cnt_acc, csem = cnt_pair
      pltpu.async_copy(ones_v, cnt_acc.at[dst_v.at[j]], csem)
    return carry

  lax.fori_loop(0, chunks, body, 0)
  if cnt_pair is not None:
    ones_v, cnt_acc, csem = cnt_pair

    def drain(j, carry):
      pltpu.make_async_copy(ones_v, cnt_acc.at[dst_v.at[0]], csem).wait()
      return carry

    lax.fori_loop(0, chunks, drain, 0)


def _make_seg1(n_nodes, width, chunks_per_tile):
  acc_rows = _acc_rows(n_nodes)
  zchunk = acc_rows // _NS
  mesh = plsc.VectorSubcoreMesh(core_axis_name="c", subcore_axis_name="s")

  @functools.partial(
      pl.kernel,
      out_type=[jax.ShapeDtypeStruct((_NC, acc_rows, width), jnp.float32),
                jax.ShapeDtypeStruct((_NC, acc_rows), jnp.float32)],
      mesh=mesh,
      scratch_types=[
          pltpu.VMEM_SHARED((acc_rows, width), jnp.float32),
          pltpu.VMEM_SHARED((acc_rows,), jnp.float32),
          pltpu.VMEM((chunks_per_tile, _CH), jnp.int32),
          pltpu.VMEM((chunks_per_tile, _CH), jnp.int32),
          pltpu.VMEM((_CH, width), jnp.float32),
          pltpu.VMEM((_CH,), jnp.float32),
          pltpu.SemaphoreType.DMA,
          pltpu.SemaphoreType.DMA,
      ],
      compiler_params=pltpu.CompilerParams(use_tc_tiling_on_sc=False, needs_layout_passes=False),
  )
  def seg1(table, srcb, dstb, zrows, zcnt, out, cnt_out,
           acc, cnt_acc, src_v, dst_v, rows_v, ones_v, sem, csem):
    c = lax.axis_index("c")
    s = lax.axis_index("s")
    wid = c * _NS + s
    pltpu.sync_copy(zrows, acc.at[pl.ds(s * zchunk, zchunk)])
    pltpu.sync_copy(zcnt, cnt_acc.at[pl.ds(s * zchunk, zchunk)])
    for i in range(_CH // 16):
      ones_v[pl.ds(i * 16, 16)] = jnp.ones((16,), jnp.float32)
    pltpu.sync_copy(srcb.at[wid], src_v)
    pltpu.sync_copy(dstb.at[wid], dst_v)
    plsc.subcore_barrier()
    _seg_loop(table, src_v, dst_v, rows_v, sem, acc, chunks_per_tile,
              cnt_pair=(ones_v, cnt_acc, csem))
    plsc.subcore_barrier()
    pltpu.sync_copy(acc.at[pl.ds(s * zchunk, zchunk)],
                    out.at[c, pl.ds(s * zchunk, zchunk)])
    pltpu.sync_copy(cnt_acc.at[pl.ds(s * zchunk, zchunk)],
                    cnt_out.at[c, pl.ds(s * zchunk, zchunk)])

  return seg1


def _make_seg2(n_nodes, width, chunks_per_tile):
  acc_rows = _acc_rows(n_nodes)
  zchunk = acc_rows // _NS
  mesh = plsc.VectorSubcoreMesh(core_axis_name="c", subcore_axis_name="s")

  @functools.partial(
      pl.kernel,
      out_type=[jax.ShapeDtypeStruct((_NC, acc_rows, width), jnp.float32),
                jax.ShapeDtypeStruct((_NC, acc_rows, width), jnp.float32)],
      mesh=mesh,
      scratch_types=[
          pltpu.VMEM_SHARED((acc_rows, width), jnp.float32),
          pltpu.VMEM((chunks_per_tile, _CH), jnp.int32),
          pltpu.VMEM((chunks_per_tile, _CH), jnp.int32),
          pltpu.VMEM((_CH, width), jnp.float32),
          pltpu.VMEM((zchunk, width), jnp.float32),
          pltpu.VMEM((zchunk, width), jnp.float32),
          pltpu.VMEM((zchunk, width), jnp.float32),
          pltpu.VMEM((zchunk,), jnp.float32),
          pltpu.VMEM((zchunk,), jnp.float32),
          pltpu.SemaphoreType.DMA,
      ],
      compiler_params=pltpu.CompilerParams(use_tc_tiling_on_sc=False, needs_layout_passes=False),
  )
  def seg2(part1, cnts, q1p, srcb, dstb, zrows, out, h_out,
           acc, src_v, dst_v, rows_v, s0_v, s1_v, q_v, cnt_v, cnt1_v, sem):
    c = lax.axis_index("c")
    s = lax.axis_index("s")
    wid = c * _NS + s
    sl = pl.ds(s * zchunk, zchunk)
    pltpu.sync_copy(zrows, acc.at[sl])
    pltpu.sync_copy(srcb.at[wid], src_v)
    pltpu.sync_copy(dstb.at[wid], dst_v)
    pltpu.sync_copy(part1.at[0, sl], s0_v)
    pltpu.sync_copy(part1.at[1, sl], s1_v)
    pltpu.sync_copy(q1p.at[sl], q_v)
    pltpu.sync_copy(cnts.at[0, sl], cnt_v)
    pltpu.sync_copy(cnts.at[1, sl], cnt1_v)

    def cbody(k, carry):
      d = pl.ds(k * 16, 16)
      cnt_v[d] = jnp.maximum(cnt_v[d] + cnt1_v[d], 1.0)
      return carry

    lax.fori_loop(0, zchunk // 16, cbody, 0)

    def hbody(i, carry):
      bc = plsc.load_gather(cnt_v, [jnp.full((16,), i, jnp.int32)])
      s0_v[i] = jnp.maximum((s0_v[i] + s1_v[i]) / bc + q_v[i], 0.0)
      return carry

    lax.fori_loop(0, zchunk, hbody, 0)
    pltpu.sync_copy(s0_v, h_out.at[c, sl])
    plsc.subcore_barrier()
    _seg_loop(h_out.at[c], src_v, dst_v, rows_v, sem, acc, chunks_per_tile)
    plsc.subcore_barrier()
    pltpu.sync_copy(acc.at[sl], out.at[c, sl])

  return seg2


def _lin_body(x_ref, w_ref, b_ref, o_ref):
  o_ref[...] = lax.dot_general(
      x_ref[...], w_ref[...], (((1,), (1,)), ((), ())),
      preferred_element_type=jnp.float32) + b_ref[...]


def _out_body(n, part2_ref, cnts_ref, h_ref, w_ref, b_ref, o_ref):
  cnt = jnp.maximum(cnts_ref[0, :n] + cnts_ref[1, :n], 1.0)[:, None]
  mean2 = (part2_ref[0, :n, :] + part2_ref[1, :n, :]) / cnt
  z = lax.dot_general(
      jnp.concatenate([mean2, h_ref[0, :n, :]], axis=1), w_ref[...],
      (((1,), (1,)), ((), ())), preferred_element_type=jnp.float32) + b_ref[...]
  z = z - jnp.max(z, axis=1, keepdims=True)
  o_ref[...] = z - jnp.log(jnp.sum(jnp.exp(z), axis=1, keepdims=True))


def kernel(x, edge_index, W1l, b1l, W1r, W2l, b2l, W2r):
  n = x.shape[0]
  e = edge_index.shape[1]
  hid = W1l.shape[0]
  out_ch = W2l.shape[0]

  src = edge_index[0].astype(jnp.int32)
  dst = edge_index[1].astype(jnp.int32)
  per = _NW * _CH
  chunks = -(-e // per)
  pad = chunks * per - e
  srcb = jnp.concatenate([src, jnp.zeros((pad,), jnp.int32)]).reshape(
      _NW, chunks, _CH)
  dstb = jnp.concatenate([dst, jnp.full((pad,), n, jnp.int32)]).reshape(
      _NW, chunks, _CH)

  npad = _acc_rows(n)
  zrows = jnp.zeros((npad // _NS, hid), jnp.float32)
  zcnt = jnp.zeros((npad // _NS,), jnp.float32)

  w1 = jnp.concatenate([W1l, W1r], axis=0)
  bias1 = jnp.concatenate([jnp.zeros((hid,), jnp.float32), b1l])[None, :]
  pq1 = pl.pallas_call(
      _lin_body,
      out_shape=jax.ShapeDtypeStruct((n, 2 * hid), jnp.float32),
  )(x, w1, bias1)

  part1, cnts = _make_seg1(n, hid, chunks)(pq1[:, :hid], srcb, dstb,
                                           zrows, zcnt)

  q1p = jnp.concatenate(
      [pq1[:, hid:], jnp.zeros((npad - n, hid), jnp.float32)])
  part2, h_out = _make_seg2(n, out_ch, chunks)(part1, cnts, q1p,
                                               srcb, dstb, zrows)

  w2 = jnp.concatenate([W2l, W2r], axis=1)
  out = pl.pallas_call(
      functools.partial(_out_body, n),
      out_shape=jax.ShapeDtypeStruct((n, out_ch), jnp.float32),
  )(part2, cnts, h_out, w2, b2l[None, :])
  return out
